# baseline (speedup 1.0000x reference)
import functools

import jax
import jax.numpy as jnp
import numpy as np
from jax import lax
from jax.experimental import pallas as pl
from jax.experimental.pallas import tpu as pltpu
from jax.experimental.pallas import tpu_sc as plsc

_CUTOFF = 20.0
_GAP = 0.1
_LOG2 = float(np.log(2.0))

_NC = 2
_NS = 16
_LANES = 16
_NW = _NC * _NS


def _sc_mesh():
    return plsc.VectorSubcoreMesh(
        core_axis_name="c", subcore_axis_name="s",
        num_cores=_NC, num_subcores=_NS)


def _dist2_body(pos_hbm, src_hbm, dst_hbm, out_hbm, pos_v, src_v, dst_v, out_v):
    c = lax.axis_index("c")
    s = lax.axis_index("s")
    wid = s * _NC + c
    chunk = src_v.shape[0]
    base = wid * chunk
    pltpu.sync_copy(pos_hbm, pos_v)
    pltpu.sync_copy(src_hbm.at[pl.ds(base, chunk)], src_v)
    pltpu.sync_copy(dst_hbm.at[pl.ds(base, chunk)], dst_v)

    def body(i, carry):
        off = i * _LANES
        si = src_v[pl.ds(off, _LANES)] * 3
        di = dst_v[pl.ds(off, _LANES)] * 3
        acc = jnp.zeros((_LANES,), jnp.float32)
        for k in range(3):
            kk = jnp.full((_LANES,), k, jnp.int32)
            a = plsc.load_gather(pos_v, [di + kk])
            b = plsc.load_gather(pos_v, [si + kk])
            d = a - b
            acc = acc + d * d
        out_v[pl.ds(off, _LANES)] = acc
        return carry

    lax.fori_loop(0, chunk // _LANES, body, 0)
    pltpu.sync_copy(out_v, out_hbm.at[pl.ds(base, chunk)])


def _dist2(positions_flat, src_pad, dst_pad):
    n3 = positions_flat.shape[0]
    e_pad = src_pad.shape[0]
    chunk = e_pad // _NW
    call = pl.kernel(
        _dist2_body,
        out_type=jax.ShapeDtypeStruct((e_pad,), jnp.float32),
        mesh=_sc_mesh(),
        scratch_types=[
            pltpu.VMEM((n3,), jnp.float32),
            pltpu.VMEM((chunk,), jnp.int32),
            pltpu.VMEM((chunk,), jnp.int32),
            pltpu.VMEM((chunk,), jnp.float32),
        ],
        compiler_params=pltpu.CompilerParams(needs_layout_passes=False),
    )
    return call(positions_flat, src_pad, dst_pad)


def _ssp(x):
    return jnp.maximum(x, 0.0) + jnp.log(1.0 + jnp.exp(-jnp.abs(x))) - _LOG2


def _mlp_body(d2_ref, cent_ref, w1_ref, b1_ref, w2_ref, b2_ref, out_ref):
    d2 = d2_ref[...]
    dist = jnp.sqrt(d2 + 1e-12)
    dd = dist - cent_ref[...]
    rbf = jnp.exp((dd * dd) * (-1.0 / _GAP))
    h = jnp.dot(rbf, w1_ref[...], preferred_element_type=jnp.float32)
    h = jnp.log(1.0 + jnp.exp(h + b1_ref[...])) - _LOG2
    h = jnp.dot(h, w2_ref[...], preferred_element_type=jnp.float32)
    out_ref[...] = _ssp(h + b2_ref[...])


def _mlp(d2, cent, w1p, b1, w2, b2, block):
    e = d2.shape[0]
    cp = cent.shape[1]
    ch = w2.shape[1]
    grid = e // block
    return pl.pallas_call(
        _mlp_body,
        grid=(grid,),
        in_specs=[
            pl.BlockSpec((block, 1), lambda i: (i, 0)),
            pl.BlockSpec((1, cp), lambda i: (0, 0)),
            pl.BlockSpec((cp, ch), lambda i: (0, 0)),
            pl.BlockSpec((1, ch), lambda i: (0, 0)),
            pl.BlockSpec((ch, ch), lambda i: (0, 0)),
            pl.BlockSpec((1, ch), lambda i: (0, 0)),
        ],
        out_specs=pl.BlockSpec((block, ch), lambda i: (i, 0)),
        out_shape=jax.ShapeDtypeStruct((e, ch), jnp.float32),
        compiler_params=pltpu.CompilerParams(
            dimension_semantics=("arbitrary",),
            vmem_limit_bytes=100 * 1024 * 1024),
    )(d2, cent, w1p, b1, w2, b2)


_NBUF = 4


def _scatter_core(h2_hbm, dst_hbm, init_hbm, out_hbm, acc_sh,
                  buf0, buf1, buf2, buf3, idx0, idx1, idx2, idx3,
                  sem0, sem1, sem2, sem3, chained):
    c = lax.axis_index("c")
    s = lax.axis_index("s")
    n_pad = acc_sh.shape[0]
    half = acc_sh.shape[1]
    rows = n_pad // _NS
    r0 = s * rows
    n_out = out_hbm.shape[0]
    tail = n_out - (_NS - 1) * rows
    col0 = c * half
    if not chained:
        pltpu.sync_copy(init_hbm, acc_sh.at[pl.ds(r0, rows)])
    else:
        @pl.when(s < _NS - 1)
        def _():
            pltpu.sync_copy(init_hbm.at[pl.ds(r0, rows), pl.ds(col0, half)],
                            acc_sh.at[pl.ds(r0, rows)])

        @pl.when(s == _NS - 1)
        def _():
            base = (_NS - 1) * rows
            pltpu.sync_copy(init_hbm.at[pl.ds(base, tail), pl.ds(col0, half)],
                            acc_sh.at[pl.ds(base, tail)])
    plsc.subcore_barrier()

    e_edges = dst_hbm.shape[0]
    epc = e_edges // _NS
    p = idx0.shape[0]
    npieces = epc // p
    e0 = s * epc
    bufs = (buf0, buf1, buf2, buf3)
    idxs = (idx0, idx1, idx2, idx3)
    sems = (sem0, sem1, sem2, sem3)

    def start_in(piece, b):
        off = e0 + piece * p
        pltpu.async_copy(dst_hbm.at[pl.ds(off, p)], idxs[b], sems[b])
        pltpu.async_copy(h2_hbm.at[pl.ds(off, p), pl.ds(col0, half)],
                         bufs[b], sems[b])

    def wait_in(piece, b):
        off = e0 + piece * p
        pltpu.make_async_copy(dst_hbm.at[pl.ds(off, p)], idxs[b], sems[b]).wait()
        pltpu.make_async_copy(h2_hbm.at[pl.ds(off, p), pl.ds(col0, half)],
                              bufs[b], sems[b]).wait()

    for b in range(_NBUF):
        start_in(b, b)

    def body(g, carry):
        for b in range(_NBUF):
            piece = g * _NBUF + b
            wait_in(piece, b)
            pltpu.sync_copy(bufs[b], acc_sh.at[idxs[b]], add=True)
            nxt = piece + _NBUF

            @pl.when(nxt < npieces)
            def _():
                start_in(nxt, b)
        return carry

    lax.fori_loop(0, npieces // _NBUF, body, 0)
    for r in range(npieces % _NBUF):
        piece = (npieces // _NBUF) * _NBUF + r
        b = piece % _NBUF
        wait_in(piece, b)
        pltpu.sync_copy(bufs[b], acc_sh.at[idxs[b]], add=True)
    plsc.subcore_barrier()

    @pl.when(s < _NS - 1)
    def _():
        pltpu.sync_copy(acc_sh.at[pl.ds(r0, rows)],
                        out_hbm.at[pl.ds(r0, rows), pl.ds(col0, half)])

    @pl.when(s == _NS - 1)
    def _():
        base = (_NS - 1) * rows
        pltpu.sync_copy(acc_sh.at[pl.ds(base, tail)],
                        out_hbm.at[pl.ds(base, tail), pl.ds(col0, half)])


def _segment_sum(h2, dst, n_nodes, init=None, piece=80):
    e, ch = h2.shape
    half = ch // _NC
    n_pad = ((n_nodes + _NS * 8 - 1) // (_NS * 8)) * (_NS * 8)
    rows = n_pad // _NS
    chained = init is not None
    body = functools.partial(_scatter_core, chained=chained)
    call = pl.kernel(
        body,
        out_type=jax.ShapeDtypeStruct((n_nodes, ch), jnp.float32),
        mesh=_sc_mesh(),
        scratch_types=[
            pltpu.VMEM_SHARED((n_pad, half), jnp.float32),
            pltpu.VMEM((piece, half), jnp.float32),
            pltpu.VMEM((piece, half), jnp.float32),
            pltpu.VMEM((piece, half), jnp.float32),
            pltpu.VMEM((piece, half), jnp.float32),
            pltpu.VMEM((piece,), jnp.int32),
            pltpu.VMEM((piece,), jnp.int32),
            pltpu.VMEM((piece,), jnp.int32),
            pltpu.VMEM((piece,), jnp.int32),
            pltpu.SemaphoreType.DMA,
            pltpu.SemaphoreType.DMA,
            pltpu.SemaphoreType.DMA,
            pltpu.SemaphoreType.DMA,
        ],
        compiler_params=pltpu.CompilerParams(needs_layout_passes=False),
    )
    if not chained:
        init = jnp.zeros((rows, half), jnp.float32)
    return call(h2, dst, init)


def kernel(positions, edge_index, weight1, bias1, weight2, bias2):
    n_nodes = positions.shape[0]
    e_edges = edge_index.shape[1]
    n_cent, ch = weight1.shape
    cp = 256

    src = edge_index[0]
    dst = edge_index[1]

    chunk = ((e_edges + _NW * _LANES - 1) // (_NW * _LANES)) * _LANES
    e_pad = chunk * _NW
    pad = e_pad - e_edges
    src_pad = jnp.concatenate([src, jnp.zeros((pad,), jnp.int32)])
    dst_pad = jnp.concatenate([dst, jnp.zeros((pad,), jnp.int32)])

    d2 = _dist2(positions.reshape(-1), src_pad, dst_pad)[:e_edges].reshape(e_edges, 1)

    centers = jnp.linspace(0.0, _CUTOFF, n_cent)
    cent = jnp.concatenate(
        [centers, jnp.full((cp - n_cent,), -1000.0, jnp.float32)]).reshape(1, cp)
    w1p = jnp.concatenate(
        [weight1, jnp.zeros((cp - n_cent, ch), jnp.float32)], axis=0)

    h2 = _mlp(d2, cent, w1p, bias1.reshape(1, ch), weight2,
              bias2.reshape(1, ch), block=3200)
    return _segment_sum(h2, dst, n_nodes)


# speedup vs baseline: 1.0430x; 1.0430x over previous
import functools

import jax
import jax.numpy as jnp
import numpy as np
from jax import lax
from jax.experimental import pallas as pl
from jax.experimental.pallas import tpu as pltpu
from jax.experimental.pallas import tpu_sc as plsc

_CUTOFF = 20.0
_GAP = 0.1
_LOG2 = float(np.log(2.0))
_LOG2E = float(np.log2(np.e))

_NC = 2
_NS = 16
_LANES = 16
_NW = _NC * _NS


def _sc_mesh():
    return plsc.VectorSubcoreMesh(
        core_axis_name="c", subcore_axis_name="s",
        num_cores=_NC, num_subcores=_NS)


def _dist2_body(pos_hbm, src_hbm, dst_hbm, out_hbm, pos_v, src_v, dst_v, out_v):
    c = lax.axis_index("c")
    s = lax.axis_index("s")
    wid = s * _NC + c
    chunk = src_v.shape[0]
    base = wid * chunk
    pltpu.sync_copy(pos_hbm, pos_v)
    pltpu.sync_copy(src_hbm.at[pl.ds(base, chunk)], src_v)
    pltpu.sync_copy(dst_hbm.at[pl.ds(base, chunk)], dst_v)

    def body(i, carry):
        off = i * _LANES
        si = src_v[pl.ds(off, _LANES)] * 3
        di = dst_v[pl.ds(off, _LANES)] * 3
        acc = jnp.zeros((_LANES,), jnp.float32)
        for k in range(3):
            kk = jnp.full((_LANES,), k, jnp.int32)
            a = plsc.load_gather(pos_v, [di + kk])
            b = plsc.load_gather(pos_v, [si + kk])
            d = a - b
            acc = acc + d * d
        out_v[pl.ds(off, _LANES)] = acc
        return carry

    lax.fori_loop(0, chunk // _LANES, body, 0)
    pltpu.sync_copy(out_v, out_hbm.at[pl.ds(base, chunk)])


def _dist2(positions_flat, src_pad, dst_pad):
    n3 = positions_flat.shape[0]
    e_pad = src_pad.shape[0]
    chunk = e_pad // _NW
    call = pl.kernel(
        _dist2_body,
        out_type=jax.ShapeDtypeStruct((e_pad,), jnp.float32),
        mesh=_sc_mesh(),
        scratch_types=[
            pltpu.VMEM((n3,), jnp.float32),
            pltpu.VMEM((chunk,), jnp.int32),
            pltpu.VMEM((chunk,), jnp.int32),
            pltpu.VMEM((chunk,), jnp.float32),
        ],
        compiler_params=pltpu.CompilerParams(needs_layout_passes=False),
    )
    return call(positions_flat, src_pad, dst_pad)


def _ssp(x):
    return (jnp.maximum(x, 0.0)
            + jnp.log(1.0 + jnp.exp2(jnp.abs(x) * -_LOG2E)) - _LOG2)


def _mlp_body(d2_ref, cent_ref, w1_ref, b1_ref, w2_ref, b2_ref, out_ref):
    d2 = d2_ref[...]
    dist = jnp.sqrt(d2 + 1e-12)
    dd = dist - cent_ref[...]
    rbf = jnp.exp2((dd * dd) * (-_LOG2E / _GAP))
    h = jnp.dot(rbf, w1_ref[...], preferred_element_type=jnp.float32)
    h = jnp.log(1.0 + jnp.exp2(h + b1_ref[...])) - _LOG2
    h = jnp.dot(h, w2_ref[...], preferred_element_type=jnp.float32)
    out_ref[...] = _ssp(h + b2_ref[...])


def _mlp(d2, cent, w1p, b1, w2, b2, block):
    e = d2.shape[0]
    cp = cent.shape[1]
    ch = w2.shape[1]
    grid = e // block
    return pl.pallas_call(
        _mlp_body,
        grid=(grid,),
        in_specs=[
            pl.BlockSpec((block, 1), lambda i: (i, 0)),
            pl.BlockSpec((1, cp), lambda i: (0, 0)),
            pl.BlockSpec((cp, ch), lambda i: (0, 0)),
            pl.BlockSpec((1, ch), lambda i: (0, 0)),
            pl.BlockSpec((ch, ch), lambda i: (0, 0)),
            pl.BlockSpec((1, ch), lambda i: (0, 0)),
        ],
        out_specs=pl.BlockSpec((block, ch), lambda i: (i, 0)),
        out_shape=jax.ShapeDtypeStruct((e, ch), jnp.float32),
        compiler_params=pltpu.CompilerParams(
            dimension_semantics=("arbitrary",),
            vmem_limit_bytes=100 * 1024 * 1024),
    )(d2, cent, w1p, b1, w2, b2)


_NBUF = 4


def _scatter_core(h2_hbm, dst_hbm, init_hbm, out_hbm, acc_sh,
                  buf0, buf1, buf2, buf3, idx0, idx1, idx2, idx3,
                  sem0, sem1, sem2, sem3, chained):
    c = lax.axis_index("c")
    s = lax.axis_index("s")
    n_pad = acc_sh.shape[0]
    half = acc_sh.shape[1]
    rows = n_pad // _NS
    r0 = s * rows
    n_out = out_hbm.shape[0]
    tail = n_out - (_NS - 1) * rows
    col0 = c * half
    if not chained:
        pltpu.sync_copy(init_hbm, acc_sh.at[pl.ds(r0, rows)])
    else:
        @pl.when(s < _NS - 1)
        def _():
            pltpu.sync_copy(init_hbm.at[pl.ds(r0, rows), pl.ds(col0, half)],
                            acc_sh.at[pl.ds(r0, rows)])

        @pl.when(s == _NS - 1)
        def _():
            base = (_NS - 1) * rows
            pltpu.sync_copy(init_hbm.at[pl.ds(base, tail), pl.ds(col0, half)],
                            acc_sh.at[pl.ds(base, tail)])
    plsc.subcore_barrier()

    e_edges = dst_hbm.shape[0]
    epc = e_edges // _NS
    p = idx0.shape[0]
    npieces = epc // p
    e0 = s * epc
    bufs = (buf0, buf1, buf2, buf3)
    idxs = (idx0, idx1, idx2, idx3)
    sems = (sem0, sem1, sem2, sem3)

    def start_in(piece, b):
        off = e0 + piece * p
        pltpu.async_copy(dst_hbm.at[pl.ds(off, p)], idxs[b], sems[b])
        pltpu.async_copy(h2_hbm.at[pl.ds(off, p), pl.ds(col0, half)],
                         bufs[b], sems[b])

    def wait_in(piece, b):
        off = e0 + piece * p
        pltpu.make_async_copy(dst_hbm.at[pl.ds(off, p)], idxs[b], sems[b]).wait()
        pltpu.make_async_copy(h2_hbm.at[pl.ds(off, p), pl.ds(col0, half)],
                              bufs[b], sems[b]).wait()

    for b in range(_NBUF):
        start_in(b, b)

    def body(g, carry):
        for b in range(_NBUF):
            piece = g * _NBUF + b
            wait_in(piece, b)
            pltpu.sync_copy(bufs[b], acc_sh.at[idxs[b]], add=True)
            nxt = piece + _NBUF

            @pl.when(nxt < npieces)
            def _():
                start_in(nxt, b)
        return carry

    lax.fori_loop(0, npieces // _NBUF, body, 0)
    for r in range(npieces % _NBUF):
        piece = (npieces // _NBUF) * _NBUF + r
        b = piece % _NBUF
        wait_in(piece, b)
        pltpu.sync_copy(bufs[b], acc_sh.at[idxs[b]], add=True)
    plsc.subcore_barrier()

    @pl.when(s < _NS - 1)
    def _():
        pltpu.sync_copy(acc_sh.at[pl.ds(r0, rows)],
                        out_hbm.at[pl.ds(r0, rows), pl.ds(col0, half)])

    @pl.when(s == _NS - 1)
    def _():
        base = (_NS - 1) * rows
        pltpu.sync_copy(acc_sh.at[pl.ds(base, tail)],
                        out_hbm.at[pl.ds(base, tail), pl.ds(col0, half)])


def _segment_sum(h2, dst, n_nodes, init=None, piece=80):
    e, ch = h2.shape
    half = ch // _NC
    n_pad = ((n_nodes + _NS * 8 - 1) // (_NS * 8)) * (_NS * 8)
    rows = n_pad // _NS
    chained = init is not None
    body = functools.partial(_scatter_core, chained=chained)
    call = pl.kernel(
        body,
        out_type=jax.ShapeDtypeStruct((n_nodes, ch), jnp.float32),
        mesh=_sc_mesh(),
        scratch_types=[
            pltpu.VMEM_SHARED((n_pad, half), jnp.float32),
            pltpu.VMEM((piece, half), jnp.float32),
            pltpu.VMEM((piece, half), jnp.float32),
            pltpu.VMEM((piece, half), jnp.float32),
            pltpu.VMEM((piece, half), jnp.float32),
            pltpu.VMEM((piece,), jnp.int32),
            pltpu.VMEM((piece,), jnp.int32),
            pltpu.VMEM((piece,), jnp.int32),
            pltpu.VMEM((piece,), jnp.int32),
            pltpu.SemaphoreType.DMA,
            pltpu.SemaphoreType.DMA,
            pltpu.SemaphoreType.DMA,
            pltpu.SemaphoreType.DMA,
        ],
        compiler_params=pltpu.CompilerParams(needs_layout_passes=False),
    )
    if not chained:
        init = jnp.zeros((rows, half), jnp.float32)
    return call(h2, dst, init)


def kernel(positions, edge_index, weight1, bias1, weight2, bias2):
    n_nodes = positions.shape[0]
    e_edges = edge_index.shape[1]
    n_cent, ch = weight1.shape
    cp = 256

    src = edge_index[0]
    dst = edge_index[1]

    chunk = ((e_edges + _NW * _LANES - 1) // (_NW * _LANES)) * _LANES
    e_pad = chunk * _NW
    pad = e_pad - e_edges
    src_pad = jnp.concatenate([src, jnp.zeros((pad,), jnp.int32)])
    dst_pad = jnp.concatenate([dst, jnp.zeros((pad,), jnp.int32)])

    d2 = _dist2(positions.reshape(-1), src_pad, dst_pad)[:e_edges].reshape(e_edges, 1)

    centers = jnp.linspace(0.0, _CUTOFF, n_cent)
    cent = jnp.concatenate(
        [centers, jnp.full((cp - n_cent,), -1000.0, jnp.float32)]).reshape(1, cp)
    w1p = jnp.concatenate(
        [weight1, jnp.zeros((cp - n_cent, ch), jnp.float32)], axis=0) * _LOG2E

    h2 = _mlp(d2, cent, w1p, bias1.reshape(1, ch) * _LOG2E, weight2,
              bias2.reshape(1, ch), block=3200)
    return _segment_sum(h2, dst, n_nodes)


# speedup vs baseline: 1.0568x; 1.0132x over previous
import functools

import jax
import jax.numpy as jnp
import numpy as np
from jax import lax
from jax.experimental import pallas as pl
from jax.experimental.pallas import tpu as pltpu
from jax.experimental.pallas import tpu_sc as plsc

_CUTOFF = 20.0
_GAP = 0.1
_LOG2 = float(np.log(2.0))
_LOG2E = float(np.log2(np.e))

_NC = 2
_NS = 16
_LANES = 16
_NW = _NC * _NS


def _sc_mesh():
    return plsc.VectorSubcoreMesh(
        core_axis_name="c", subcore_axis_name="s",
        num_cores=_NC, num_subcores=_NS)


def _dist2_body(pos_hbm, src_hbm, dst_hbm, out_hbm, pos_v, src_v, dst_v, out_v):
    c = lax.axis_index("c")
    s = lax.axis_index("s")
    wid = s * _NC + c
    chunk = src_v.shape[0]
    base = wid * chunk
    pltpu.sync_copy(pos_hbm, pos_v)
    pltpu.sync_copy(src_hbm.at[pl.ds(base, chunk)], src_v)
    pltpu.sync_copy(dst_hbm.at[pl.ds(base, chunk)], dst_v)

    def body(i, carry):
        off = i * _LANES
        si = src_v[pl.ds(off, _LANES)] * 3
        di = dst_v[pl.ds(off, _LANES)] * 3
        acc = jnp.zeros((_LANES,), jnp.float32)
        for k in range(3):
            kk = jnp.full((_LANES,), k, jnp.int32)
            a = plsc.load_gather(pos_v, [di + kk])
            b = plsc.load_gather(pos_v, [si + kk])
            d = a - b
            acc = acc + d * d
        out_v[pl.ds(off, _LANES)] = acc
        return carry

    lax.fori_loop(0, chunk // _LANES, body, 0)
    pltpu.sync_copy(out_v, out_hbm.at[pl.ds(base, chunk)])


def _dist2(positions_flat, src_pad, dst_pad):
    n3 = positions_flat.shape[0]
    e_pad = src_pad.shape[0]
    chunk = e_pad // _NW
    call = pl.kernel(
        _dist2_body,
        out_type=jax.ShapeDtypeStruct((e_pad,), jnp.float32),
        mesh=_sc_mesh(),
        scratch_types=[
            pltpu.VMEM((n3,), jnp.float32),
            pltpu.VMEM((chunk,), jnp.int32),
            pltpu.VMEM((chunk,), jnp.int32),
            pltpu.VMEM((chunk,), jnp.float32),
        ],
        compiler_params=pltpu.CompilerParams(needs_layout_passes=False),
    )
    return call(positions_flat, src_pad, dst_pad)


def _ssp(x):
    return (jnp.maximum(x, 0.0)
            + jnp.log(1.0 + jnp.exp2(jnp.abs(x) * -_LOG2E)) - _LOG2)


def _mlp_body(d2_ref, cent_ref, w1_ref, b1_ref, w2_ref, b2_ref, out_ref):
    d2 = d2_ref[...]
    dist = jnp.sqrt(d2 + 1e-12)
    dd = dist - cent_ref[...]
    rbf = jnp.exp2((dd * dd) * (-_LOG2E / _GAP))
    h = jnp.dot(rbf, w1_ref[...], preferred_element_type=jnp.float32)
    h = jnp.log(1.0 + jnp.exp2(h + b1_ref[...]))
    h = jnp.dot(h, w2_ref[...], preferred_element_type=jnp.float32)
    out_ref[...] = _ssp(h + b2_ref[...])


def _mlp(d2, cent, w1p, b1, w2, b2, block):
    e = d2.shape[0]
    cp = cent.shape[1]
    ch = w2.shape[1]
    grid = e // block
    return pl.pallas_call(
        _mlp_body,
        grid=(grid,),
        in_specs=[
            pl.BlockSpec((block, 1), lambda i: (i, 0)),
            pl.BlockSpec((1, cp), lambda i: (0, 0)),
            pl.BlockSpec((cp, ch), lambda i: (0, 0)),
            pl.BlockSpec((1, ch), lambda i: (0, 0)),
            pl.BlockSpec((ch, ch), lambda i: (0, 0)),
            pl.BlockSpec((1, ch), lambda i: (0, 0)),
        ],
        out_specs=pl.BlockSpec((block, ch), lambda i: (i, 0)),
        out_shape=jax.ShapeDtypeStruct((e, ch), jnp.float32),
        compiler_params=pltpu.CompilerParams(
            dimension_semantics=("arbitrary",),
            vmem_limit_bytes=100 * 1024 * 1024),
    )(d2, cent, w1p, b1, w2, b2)


_NBUF = 4


def _scatter_core(h2_hbm, dst_hbm, init_hbm, out_hbm, acc_sh,
                  buf0, buf1, buf2, buf3, idx0, idx1, idx2, idx3,
                  sem0, sem1, sem2, sem3, chained):
    c = lax.axis_index("c")
    s = lax.axis_index("s")
    n_pad = acc_sh.shape[0]
    half = acc_sh.shape[1]
    rows = n_pad // _NS
    r0 = s * rows
    n_out = out_hbm.shape[0]
    tail = n_out - (_NS - 1) * rows
    col0 = c * half
    if not chained:
        pltpu.sync_copy(init_hbm, acc_sh.at[pl.ds(r0, rows)])
    else:
        @pl.when(s < _NS - 1)
        def _():
            pltpu.sync_copy(init_hbm.at[pl.ds(r0, rows), pl.ds(col0, half)],
                            acc_sh.at[pl.ds(r0, rows)])

        @pl.when(s == _NS - 1)
        def _():
            base = (_NS - 1) * rows
            pltpu.sync_copy(init_hbm.at[pl.ds(base, tail), pl.ds(col0, half)],
                            acc_sh.at[pl.ds(base, tail)])
    plsc.subcore_barrier()

    e_edges = dst_hbm.shape[0]
    epc = e_edges // _NS
    p = idx0.shape[0]
    npieces = epc // p
    e0 = s * epc
    bufs = (buf0, buf1, buf2, buf3)
    idxs = (idx0, idx1, idx2, idx3)
    sems = (sem0, sem1, sem2, sem3)

    def start_in(piece, b):
        off = e0 + piece * p
        pltpu.async_copy(dst_hbm.at[pl.ds(off, p)], idxs[b], sems[b])
        pltpu.async_copy(h2_hbm.at[pl.ds(off, p), pl.ds(col0, half)],
                         bufs[b], sems[b])

    def wait_in(piece, b):
        off = e0 + piece * p
        pltpu.make_async_copy(dst_hbm.at[pl.ds(off, p)], idxs[b], sems[b]).wait()
        pltpu.make_async_copy(h2_hbm.at[pl.ds(off, p), pl.ds(col0, half)],
                              bufs[b], sems[b]).wait()

    for b in range(_NBUF):
        start_in(b, b)

    def body(g, carry):
        for b in range(_NBUF):
            piece = g * _NBUF + b
            wait_in(piece, b)
            pltpu.sync_copy(bufs[b], acc_sh.at[idxs[b]], add=True)
            nxt = piece + _NBUF

            @pl.when(nxt < npieces)
            def _():
                start_in(nxt, b)
        return carry

    lax.fori_loop(0, npieces // _NBUF, body, 0)
    for r in range(npieces % _NBUF):
        piece = (npieces // _NBUF) * _NBUF + r
        b = piece % _NBUF
        wait_in(piece, b)
        pltpu.sync_copy(bufs[b], acc_sh.at[idxs[b]], add=True)
    plsc.subcore_barrier()

    @pl.when(s < _NS - 1)
    def _():
        pltpu.sync_copy(acc_sh.at[pl.ds(r0, rows)],
                        out_hbm.at[pl.ds(r0, rows), pl.ds(col0, half)])

    @pl.when(s == _NS - 1)
    def _():
        base = (_NS - 1) * rows
        pltpu.sync_copy(acc_sh.at[pl.ds(base, tail)],
                        out_hbm.at[pl.ds(base, tail), pl.ds(col0, half)])


def _segment_sum(h2, dst, n_nodes, init=None, piece=80):
    e, ch = h2.shape
    half = ch // _NC
    n_pad = ((n_nodes + _NS * 8 - 1) // (_NS * 8)) * (_NS * 8)
    rows = n_pad // _NS
    chained = init is not None
    body = functools.partial(_scatter_core, chained=chained)
    call = pl.kernel(
        body,
        out_type=jax.ShapeDtypeStruct((n_nodes, ch), jnp.float32),
        mesh=_sc_mesh(),
        scratch_types=[
            pltpu.VMEM_SHARED((n_pad, half), jnp.float32),
            pltpu.VMEM((piece, half), jnp.float32),
            pltpu.VMEM((piece, half), jnp.float32),
            pltpu.VMEM((piece, half), jnp.float32),
            pltpu.VMEM((piece, half), jnp.float32),
            pltpu.VMEM((piece,), jnp.int32),
            pltpu.VMEM((piece,), jnp.int32),
            pltpu.VMEM((piece,), jnp.int32),
            pltpu.VMEM((piece,), jnp.int32),
            pltpu.SemaphoreType.DMA,
            pltpu.SemaphoreType.DMA,
            pltpu.SemaphoreType.DMA,
            pltpu.SemaphoreType.DMA,
        ],
        compiler_params=pltpu.CompilerParams(needs_layout_passes=False),
    )
    if not chained:
        init = jnp.zeros((rows, half), jnp.float32)
    return call(h2, dst, init)


def kernel(positions, edge_index, weight1, bias1, weight2, bias2):
    n_nodes = positions.shape[0]
    e_edges = edge_index.shape[1]
    n_cent, ch = weight1.shape
    cp = 256

    src = edge_index[0]
    dst = edge_index[1]

    chunk = ((e_edges + _NW * _LANES - 1) // (_NW * _LANES)) * _LANES
    e_pad = chunk * _NW
    pad = e_pad - e_edges
    src_pad = jnp.concatenate([src, jnp.zeros((pad,), jnp.int32)])
    dst_pad = jnp.concatenate([dst, jnp.zeros((pad,), jnp.int32)])

    d2 = _dist2(positions.reshape(-1), src_pad, dst_pad)[:e_edges].reshape(e_edges, 1)

    centers = jnp.linspace(0.0, _CUTOFF, n_cent)
    cent = jnp.concatenate(
        [centers, jnp.full((cp - n_cent,), -1000.0, jnp.float32)]).reshape(1, cp)
    w1p = jnp.concatenate(
        [weight1, jnp.zeros((cp - n_cent, ch), jnp.float32)], axis=0) * _LOG2E

    b2f = (bias2 - _LOG2 * jnp.sum(weight2, axis=0)).reshape(1, ch)
    h2 = _mlp(d2, cent, w1p, bias1.reshape(1, ch) * _LOG2E, weight2,
              b2f, block=3200)
    return _segment_sum(h2, dst, n_nodes)
